# baseline (speedup 1.0000x reference)
import jax
import jax.numpy as jnp
from jax.experimental import pallas as pl
from jax.experimental.pallas import tpu as pltpu
from jax.experimental.pallas import tpu_sc as plsc

_E = 8
_H = 768
_H2 = _H // 2
_T = 4096
_RT = 256
_MT = 256
_NBUF = _T * 2 + _E * _MT
_SCW = 128


def _router_kernel(x_ref, rw_ref, rb_ref, xl_ref, xr_ref, p_ref, eid_ref,
                   rank_ref, counts_ref, carry_ref):
    t = pl.program_id(0)

    @pl.when(t == 0)
    def _init():
        carry_ref[...] = jnp.zeros_like(carry_ref)

    x = x_ref[...]
    scores = jnp.dot(x, rw_ref[...], preferred_element_type=jnp.float32)
    scores = scores + rb_ref[...]
    m = jnp.max(scores, axis=-1, keepdims=True)
    ex = jnp.exp(scores - m)
    probs = ex / jnp.sum(ex, axis=-1, keepdims=True)

    i0 = jnp.argmax(probs, axis=-1).reshape(-1, 1)
    p0 = jnp.max(probs, axis=-1, keepdims=True)
    iota = jax.lax.broadcasted_iota(jnp.int32, probs.shape, 1)
    masked = jnp.where(iota == i0, probs - 2.0, probs)
    i1 = jnp.argmax(masked, axis=-1).reshape(-1, 1)
    p1 = jnp.max(masked, axis=-1, keepdims=True)
    denom = p0 + p1

    oh0 = (iota == i0).astype(jnp.float32)
    oh1 = (iota == i1).astype(jnp.float32)
    r_iota = jax.lax.broadcasted_iota(jnp.int32, (_RT, _RT), 0)
    c_iota = jax.lax.broadcasted_iota(jnp.int32, (_RT, _RT), 1)
    tri = (r_iota > c_iota).astype(jnp.float32)
    s0 = jnp.dot(tri, oh0, preferred_element_type=jnp.float32)
    c0 = jnp.sum(oh0, axis=0, keepdims=True)
    s1 = c0 + jnp.dot(tri, oh1, preferred_element_type=jnp.float32)
    c1 = jnp.sum(oh1, axis=0, keepdims=True)

    carry = carry_ref[...]
    rank0 = jnp.sum(oh0 * (carry + s0), axis=-1, keepdims=True)
    rank1 = jnp.sum(oh1 * (carry + s1), axis=-1, keepdims=True)
    carry = carry + c0 + c1
    carry_ref[...] = carry
    counts_ref[...] = carry.astype(jnp.int32)

    xl_ref[...] = x[:, :_H2]
    xr_ref[...] = x[:, _H2:]
    p_ref[...] = jnp.concatenate([p0 / denom, p1 / denom], axis=1)
    eid_ref[...] = jnp.concatenate([i0, i1], axis=1)
    rank_ref[...] = jnp.concatenate([rank0, rank1], axis=1).astype(jnp.int32)


def _posmap_kernel(eid_ref, rank_ref, counts_ref, pos_ref, te_ref):
    counts = counts_ref[...].astype(jnp.float32)
    pad = jnp.ceil(counts / _MT) * _MT
    r8 = jax.lax.broadcasted_iota(jnp.int32, (_E, _E), 0)
    c8 = jax.lax.broadcasted_iota(jnp.int32, (_E, _E), 1)
    ltri = (r8 < c8).astype(jnp.float32)
    off = jnp.dot(pad, ltri, preferred_element_type=jnp.float32)

    eid = eid_ref[...]
    iota_a = jax.lax.broadcasted_iota(jnp.int32, (_T, _E), 1)
    oha = (eid[:, 0:1] == iota_a).astype(jnp.float32)
    ohb = (eid[:, 1:2] == iota_a).astype(jnp.float32)
    off_col = off.reshape(_E, 1)
    base_a = jnp.dot(oha, off_col, preferred_element_type=jnp.float32)
    base_b = jnp.dot(ohb, off_col, preferred_element_type=jnp.float32)
    pos = rank_ref[...] + jnp.concatenate([base_a, base_b], axis=1).astype(jnp.int32)
    pos_ref[...] = pos.T

    n_tiles = _NBUF // _MT
    tvals = (jax.lax.broadcasted_iota(jnp.int32, (n_tiles, 1), 0) * _MT).astype(
        jnp.float32
    )
    te = jnp.sum((off <= tvals).astype(jnp.int32), axis=-1, keepdims=True) - 1
    te_ref[...] = te


def _sc_scatter(xh, pos):
    mesh = plsc.VectorSubcoreMesh(core_axis_name="core",
                                  subcore_axis_name="subcore")

    @pl.kernel(out_type=jax.ShapeDtypeStruct((_NBUF, _H2), jnp.float32),
               mesh=mesh)
    def k(x_hbm, i_hbm, o_hbm):
        def body(x_vmem, i_vmem):
            pltpu.sync_copy(x_vmem, o_hbm.at[i_vmem.at[0]])

        pltpu.emit_pipeline(
            body,
            grid=(2, _T // _SCW),
            in_specs=[
                pl.BlockSpec((_SCW, _H2), index_map=lambda k, i: (i, 0)),
                pl.BlockSpec((1, _SCW), index_map=lambda k, i: (k, i)),
            ],
            out_specs=[],
            core_axis_name=("core", "subcore"),
            dimension_semantics=(pltpu.PARALLEL, pltpu.PARALLEL),
        )(x_hbm, i_hbm)

    return k(xh, pos)


def _sc_gather(yh, pos_flat):
    mesh = plsc.VectorSubcoreMesh(core_axis_name="core",
                                  subcore_axis_name="subcore")

    @pl.kernel(out_type=jax.ShapeDtypeStruct((2 * _T, _H2), jnp.float32),
               mesh=mesh)
    def k(y_hbm, i_hbm, o_hbm):
        def body(i_vmem, o_vmem):
            pltpu.sync_copy(y_hbm.at[i_vmem.at[0]], o_vmem)

        pltpu.emit_pipeline(
            body,
            grid=(2 * _T // _SCW,),
            in_specs=[pl.BlockSpec((1, _SCW), index_map=lambda i: (0, i))],
            out_specs=[pl.BlockSpec((_SCW, _H2), index_map=lambda i: (i, 0))],
            core_axis_name=("core", "subcore"),
            dimension_semantics=(pltpu.PARALLEL,),
        )(i_hbm, o_hbm)

    return k(yh, pos_flat)


def _matmul_kernel(te_ref, xgl_ref, xgr_ref, ew_ref, eb_ref, yl_ref, yr_ref,
                   wb_ref):
    t = pl.program_id(0)

    @pl.when(t == 0)
    def _cast_weights():
        wb_ref[...] = ew_ref[...].astype(jnp.bfloat16)

    e = te_ref[t, 0]
    xb = jnp.concatenate([xgl_ref[...], xgr_ref[...]], axis=1).astype(
        jnp.bfloat16
    )
    y = jnp.dot(xb, wb_ref[e], preferred_element_type=jnp.float32)
    y = y + eb_ref[pl.ds(e, 1), :]
    yl_ref[...] = y[:, :_H2]
    yr_ref[...] = y[:, _H2:]


def _combine_kernel(yl0_ref, yl1_ref, yr0_ref, yr1_ref, p_ref, out_ref):
    w0 = p_ref[:, 0:1]
    w1 = p_ref[:, 1:2]
    left = w0 * yl0_ref[0] + w1 * yl1_ref[0]
    right = w0 * yr0_ref[0] + w1 * yr1_ref[0]
    out_ref[...] = jnp.concatenate([left, right], axis=1)


def _router(flat_x, router_w, rb2):
    return pl.pallas_call(
        _router_kernel,
        grid=(_T // _RT,),
        in_specs=[
            pl.BlockSpec((_RT, _H), lambda t: (t, 0)),
            pl.BlockSpec((_H, _E), lambda t: (0, 0)),
            pl.BlockSpec((1, _E), lambda t: (0, 0)),
        ],
        out_specs=[
            pl.BlockSpec((_RT, _H2), lambda t: (t, 0)),
            pl.BlockSpec((_RT, _H2), lambda t: (t, 0)),
            pl.BlockSpec((_RT, 2), lambda t: (t, 0)),
            pl.BlockSpec((_RT, 2), lambda t: (t, 0)),
            pl.BlockSpec((_RT, 2), lambda t: (t, 0)),
            pl.BlockSpec((1, _E), lambda t: (0, 0)),
        ],
        out_shape=[
            jax.ShapeDtypeStruct((_T, _H2), jnp.float32),
            jax.ShapeDtypeStruct((_T, _H2), jnp.float32),
            jax.ShapeDtypeStruct((_T, 2), jnp.float32),
            jax.ShapeDtypeStruct((_T, 2), jnp.int32),
            jax.ShapeDtypeStruct((_T, 2), jnp.int32),
            jax.ShapeDtypeStruct((1, _E), jnp.int32),
        ],
        scratch_shapes=[pltpu.VMEM((1, _E), jnp.float32)],
    )(flat_x, router_w, rb2)


def _posmap(eid, rank, counts):
    n_tiles = _NBUF // _MT
    return pl.pallas_call(
        _posmap_kernel,
        grid=(1,),
        in_specs=[
            pl.BlockSpec((_T, 2), lambda t: (0, 0)),
            pl.BlockSpec((_T, 2), lambda t: (0, 0)),
            pl.BlockSpec((1, _E), lambda t: (0, 0)),
        ],
        out_specs=[
            pl.BlockSpec((2, _T), lambda t: (0, 0)),
            pl.BlockSpec((n_tiles, 1), lambda t: (0, 0)),
        ],
        out_shape=[
            jax.ShapeDtypeStruct((2, _T), jnp.int32),
            jax.ShapeDtypeStruct((n_tiles, 1), jnp.int32),
        ],
    )(eid, rank, counts)


def _grouped_matmul(te, xgl, xgr, expert_w, expert_b):
    grid_spec = pltpu.PrefetchScalarGridSpec(
        num_scalar_prefetch=1,
        grid=(_NBUF // _MT,),
        in_specs=[
            pl.BlockSpec((_MT, _H2), lambda t, te_ref: (t, 0)),
            pl.BlockSpec((_MT, _H2), lambda t, te_ref: (t, 0)),
            pl.BlockSpec((_E, _H, _H), lambda t, te_ref: (0, 0, 0)),
            pl.BlockSpec((_E, _H), lambda t, te_ref: (0, 0)),
        ],
        out_specs=[
            pl.BlockSpec((_MT, _H2), lambda t, te_ref: (t, 0)),
            pl.BlockSpec((_MT, _H2), lambda t, te_ref: (t, 0)),
        ],
        scratch_shapes=[pltpu.VMEM((_E, _H, _H), jnp.bfloat16)],
    )
    return pl.pallas_call(
        _matmul_kernel,
        grid_spec=grid_spec,
        out_shape=[
            jax.ShapeDtypeStruct((_NBUF, _H2), jnp.float32),
            jax.ShapeDtypeStruct((_NBUF, _H2), jnp.float32),
        ],
    )(te, xgl, xgr, expert_w, expert_b)


def _combine(ygl, ygr, p):
    ct = 512
    return pl.pallas_call(
        _combine_kernel,
        grid=(_T // ct,),
        in_specs=[
            pl.BlockSpec((1, ct, _H2), lambda t: (0, t, 0)),
            pl.BlockSpec((1, ct, _H2), lambda t: (1, t, 0)),
            pl.BlockSpec((1, ct, _H2), lambda t: (0, t, 0)),
            pl.BlockSpec((1, ct, _H2), lambda t: (1, t, 0)),
            pl.BlockSpec((ct, 2), lambda t: (t, 0)),
        ],
        out_specs=pl.BlockSpec((ct, _H), lambda t: (t, 0)),
        out_shape=jax.ShapeDtypeStruct((_T, _H), jnp.float32),
    )(ygl, ygl, ygr, ygr, p)


def kernel(x, router_w, router_b, expert_w, expert_b):
    b, s, h = x.shape
    flat_x = x.reshape(_T, _H)
    rb2 = router_b.reshape(1, -1)

    xl, xr, p, eid, rank, counts = _router(flat_x, router_w, rb2)
    pos, te = _posmap(eid, rank, counts)
    xgl = _sc_scatter(xl, pos)
    xgr = _sc_scatter(xr, pos)
    yl, yr = _grouped_matmul(te, xgl, xgr, expert_w, expert_b)
    pos_flat = pos.reshape(1, 2 * _T)
    ygl = _sc_gather(yl, pos_flat)
    ygr = _sc_gather(yr, pos_flat)
    out = _combine(ygl.reshape(2, _T, _H2), ygr.reshape(2, _T, _H2), p)
    return out.reshape(b, s, h)


# speedup vs baseline: 1.0465x; 1.0465x over previous
import jax
import jax.numpy as jnp
from jax.experimental import pallas as pl
from jax.experimental.pallas import tpu as pltpu
from jax.experimental.pallas import tpu_sc as plsc

_E = 8
_H = 768
_H2 = _H // 2
_T = 4096
_RT = 256
_MT = 256
_NBUF = _T * 2 + _E * _MT
_SCW = 128


def _router_kernel(x_ref, rw_ref, rb_ref, xlr_ref, p_ref, eid_ref,
                   rank_ref, counts_ref, carry_ref):
    t = pl.program_id(0)

    @pl.when(t == 0)
    def _init():
        carry_ref[...] = jnp.zeros_like(carry_ref)

    x = x_ref[...]
    scores = jnp.dot(x, rw_ref[...], preferred_element_type=jnp.float32)
    scores = scores + rb_ref[...]
    m = jnp.max(scores, axis=-1, keepdims=True)
    ex = jnp.exp(scores - m)
    probs = ex / jnp.sum(ex, axis=-1, keepdims=True)

    i0 = jnp.argmax(probs, axis=-1).reshape(-1, 1)
    p0 = jnp.max(probs, axis=-1, keepdims=True)
    iota = jax.lax.broadcasted_iota(jnp.int32, probs.shape, 1)
    masked = jnp.where(iota == i0, probs - 2.0, probs)
    i1 = jnp.argmax(masked, axis=-1).reshape(-1, 1)
    p1 = jnp.max(masked, axis=-1, keepdims=True)
    denom = p0 + p1

    oh0 = (iota == i0).astype(jnp.float32)
    oh1 = (iota == i1).astype(jnp.float32)
    r_iota = jax.lax.broadcasted_iota(jnp.int32, (_RT, _RT), 0)
    c_iota = jax.lax.broadcasted_iota(jnp.int32, (_RT, _RT), 1)
    tri = (r_iota > c_iota).astype(jnp.float32)
    s0 = jnp.dot(tri, oh0, preferred_element_type=jnp.float32)
    c0 = jnp.sum(oh0, axis=0, keepdims=True)
    s1 = c0 + jnp.dot(tri, oh1, preferred_element_type=jnp.float32)
    c1 = jnp.sum(oh1, axis=0, keepdims=True)

    carry = carry_ref[...]
    rank0 = jnp.sum(oh0 * (carry + s0), axis=-1, keepdims=True)
    rank1 = jnp.sum(oh1 * (carry + s1), axis=-1, keepdims=True)
    carry = carry + c0 + c1
    carry_ref[...] = carry
    counts_ref[...] = carry.astype(jnp.int32)

    xlr_ref[0] = x[:, :_H2]
    xlr_ref[1] = x[:, _H2:]
    p_ref[...] = jnp.concatenate([p0 / denom, p1 / denom], axis=1)
    eid_ref[...] = jnp.concatenate([i0, i1], axis=1)
    rank_ref[...] = jnp.concatenate([rank0, rank1], axis=1).astype(jnp.int32)


def _posmap_kernel(eid_ref, rank_ref, counts_ref, pos_ref, te_ref):
    counts = counts_ref[...].astype(jnp.float32)
    pad = jnp.ceil(counts / _MT) * _MT
    r8 = jax.lax.broadcasted_iota(jnp.int32, (_E, _E), 0)
    c8 = jax.lax.broadcasted_iota(jnp.int32, (_E, _E), 1)
    ltri = (r8 < c8).astype(jnp.float32)
    off = jnp.dot(pad, ltri, preferred_element_type=jnp.float32)

    eid = eid_ref[...]
    iota_a = jax.lax.broadcasted_iota(jnp.int32, (_T, _E), 1)
    oha = (eid[:, 0:1] == iota_a).astype(jnp.float32)
    ohb = (eid[:, 1:2] == iota_a).astype(jnp.float32)
    off_col = off.reshape(_E, 1)
    base_a = jnp.dot(oha, off_col, preferred_element_type=jnp.float32)
    base_b = jnp.dot(ohb, off_col, preferred_element_type=jnp.float32)
    pos = rank_ref[...] + jnp.concatenate([base_a, base_b], axis=1).astype(jnp.int32)
    post = pos.T
    pos_ref[...] = jnp.concatenate([post, post + _NBUF], axis=0)

    n_tiles = _NBUF // _MT
    tvals = (jax.lax.broadcasted_iota(jnp.int32, (n_tiles, 1), 0) * _MT).astype(
        jnp.float32
    )
    te = jnp.sum((off <= tvals).astype(jnp.int32), axis=-1, keepdims=True) - 1
    te_ref[...] = te


def _sc_scatter(xlr, pos4):
    mesh = plsc.VectorSubcoreMesh(core_axis_name="core",
                                  subcore_axis_name="subcore")

    @pl.kernel(out_type=jax.ShapeDtypeStruct((2 * _NBUF, _H2), jnp.float32),
               mesh=mesh)
    def k(x_hbm, i_hbm, o_hbm):
        def body(x_vmem, i_vmem):
            pltpu.sync_copy(x_vmem.at[0], o_hbm.at[i_vmem.at[0]])

        pltpu.emit_pipeline(
            body,
            grid=(2, 2, _T // _SCW),
            in_specs=[
                pl.BlockSpec((1, _SCW, _H2), index_map=lambda h, k, i: (h, i, 0)),
                pl.BlockSpec((1, _SCW), index_map=lambda h, k, i: (2 * h + k, i)),
            ],
            out_specs=[],
            core_axis_name=("core", "subcore"),
            dimension_semantics=(pltpu.PARALLEL, pltpu.PARALLEL, pltpu.PARALLEL),
        )(x_hbm, i_hbm)

    return k(xlr, pos4)


def _sc_gather(y2flat, pos2):
    mesh = plsc.VectorSubcoreMesh(core_axis_name="core",
                                  subcore_axis_name="subcore")

    @pl.kernel(out_type=jax.ShapeDtypeStruct((2, 2 * _T, _H2), jnp.float32),
               mesh=mesh)
    def k(y_hbm, i_hbm, o_hbm):
        def body(i_vmem, o_vmem):
            pltpu.sync_copy(y_hbm.at[i_vmem.at[0]], o_vmem.at[0])

        pltpu.emit_pipeline(
            body,
            grid=(2, 2 * _T // _SCW),
            in_specs=[pl.BlockSpec((1, _SCW), index_map=lambda h, i: (h, i))],
            out_specs=[
                pl.BlockSpec((1, _SCW, _H2), index_map=lambda h, i: (h, i, 0))
            ],
            core_axis_name=("core", "subcore"),
            dimension_semantics=(pltpu.PARALLEL, pltpu.PARALLEL),
        )(i_hbm, o_hbm)

    return k(y2flat, pos2)


def _matmul_kernel(te_ref, xgl_ref, xgr_ref, ew_ref, eb_ref, y2_ref, wb_ref):
    t = pl.program_id(0)

    @pl.when(t == 0)
    def _cast_weights():
        wb_ref[...] = ew_ref[...].astype(jnp.bfloat16)

    e = te_ref[t, 0]
    xb = jnp.concatenate([xgl_ref[0], xgr_ref[0]], axis=1).astype(jnp.bfloat16)
    y = jnp.dot(xb, wb_ref[e], preferred_element_type=jnp.float32)
    y = y + eb_ref[pl.ds(e, 1), :]
    y2_ref[0] = y[:, :_H2]
    y2_ref[1] = y[:, _H2:]


def _combine_kernel(yl0_ref, yl1_ref, yr0_ref, yr1_ref, p_ref, out_ref):
    w0 = p_ref[:, 0:1]
    w1 = p_ref[:, 1:2]
    left = w0 * yl0_ref[0] + w1 * yl1_ref[0]
    right = w0 * yr0_ref[0] + w1 * yr1_ref[0]
    out_ref[...] = jnp.concatenate([left, right], axis=1)


def _router(flat_x, router_w, rb2):
    return pl.pallas_call(
        _router_kernel,
        grid=(_T // _RT,),
        in_specs=[
            pl.BlockSpec((_RT, _H), lambda t: (t, 0)),
            pl.BlockSpec((_H, _E), lambda t: (0, 0)),
            pl.BlockSpec((1, _E), lambda t: (0, 0)),
        ],
        out_specs=[
            pl.BlockSpec((2, _RT, _H2), lambda t: (0, t, 0)),
            pl.BlockSpec((_RT, 2), lambda t: (t, 0)),
            pl.BlockSpec((_RT, 2), lambda t: (t, 0)),
            pl.BlockSpec((_RT, 2), lambda t: (t, 0)),
            pl.BlockSpec((1, _E), lambda t: (0, 0)),
        ],
        out_shape=[
            jax.ShapeDtypeStruct((2, _T, _H2), jnp.float32),
            jax.ShapeDtypeStruct((_T, 2), jnp.float32),
            jax.ShapeDtypeStruct((_T, 2), jnp.int32),
            jax.ShapeDtypeStruct((_T, 2), jnp.int32),
            jax.ShapeDtypeStruct((1, _E), jnp.int32),
        ],
        scratch_shapes=[pltpu.VMEM((1, _E), jnp.float32)],
    )(flat_x, router_w, rb2)


def _posmap(eid, rank, counts):
    n_tiles = _NBUF // _MT
    return pl.pallas_call(
        _posmap_kernel,
        grid=(1,),
        in_specs=[
            pl.BlockSpec((_T, 2), lambda t: (0, 0)),
            pl.BlockSpec((_T, 2), lambda t: (0, 0)),
            pl.BlockSpec((1, _E), lambda t: (0, 0)),
        ],
        out_specs=[
            pl.BlockSpec((4, _T), lambda t: (0, 0)),
            pl.BlockSpec((n_tiles, 1), lambda t: (0, 0)),
        ],
        out_shape=[
            jax.ShapeDtypeStruct((4, _T), jnp.int32),
            jax.ShapeDtypeStruct((n_tiles, 1), jnp.int32),
        ],
    )(eid, rank, counts)


def _grouped_matmul(te, xg2, expert_w, expert_b):
    xg3 = xg2.reshape(2, _NBUF, _H2)
    grid_spec = pltpu.PrefetchScalarGridSpec(
        num_scalar_prefetch=1,
        grid=(_NBUF // _MT,),
        in_specs=[
            pl.BlockSpec((1, _MT, _H2), lambda t, te_ref: (0, t, 0)),
            pl.BlockSpec((1, _MT, _H2), lambda t, te_ref: (1, t, 0)),
            pl.BlockSpec((_E, _H, _H), lambda t, te_ref: (0, 0, 0)),
            pl.BlockSpec((_E, _H), lambda t, te_ref: (0, 0)),
        ],
        out_specs=pl.BlockSpec((2, _MT, _H2), lambda t, te_ref: (0, t, 0)),
        scratch_shapes=[pltpu.VMEM((_E, _H, _H), jnp.bfloat16)],
    )
    return pl.pallas_call(
        _matmul_kernel,
        grid_spec=grid_spec,
        out_shape=jax.ShapeDtypeStruct((2, _NBUF, _H2), jnp.float32),
    )(te, xg3, xg3, expert_w, expert_b)


def _combine(yg, p):
    ct = 512
    kb = _T // ct
    return pl.pallas_call(
        _combine_kernel,
        grid=(kb,),
        in_specs=[
            pl.BlockSpec((1, ct, _H2), lambda t: (0, t, 0)),
            pl.BlockSpec((1, ct, _H2), lambda t: (0, t + kb, 0)),
            pl.BlockSpec((1, ct, _H2), lambda t: (1, t, 0)),
            pl.BlockSpec((1, ct, _H2), lambda t: (1, t + kb, 0)),
            pl.BlockSpec((ct, 2), lambda t: (t, 0)),
        ],
        out_specs=pl.BlockSpec((ct, _H), lambda t: (t, 0)),
        out_shape=jax.ShapeDtypeStruct((_T, _H), jnp.float32),
    )(yg, yg, yg, yg, p)


def kernel(x, router_w, router_b, expert_w, expert_b):
    b, s, h = x.shape
    flat_x = x.reshape(_T, _H)
    rb2 = router_b.reshape(1, -1)

    xlr, p, eid, rank, counts = _router(flat_x, router_w, rb2)
    pos4, te = _posmap(eid, rank, counts)
    xg2 = _sc_scatter(xlr, pos4)
    y2 = _grouped_matmul(te, xg2, expert_w, expert_b)
    yg = _sc_gather(y2.reshape(2 * _NBUF, _H2), pos4.reshape(2, 2 * _T))
    out = _combine(yg, p)
    return out.reshape(b, s, h)


# speedup vs baseline: 2.7093x; 2.5889x over previous
import jax
import jax.numpy as jnp
from jax.experimental import pallas as pl
from jax.experimental.pallas import tpu as pltpu

_NUM_EXPERTS = 8
_TILE = 512


def _moe_tile_kernel(x_ref, rw_ref, rb_ref, ew_ref, eb_ref, out_ref, wb_ref):
    @pl.when(pl.program_id(0) == 0)
    def _cast_weights():
        wb_ref[...] = ew_ref[...].astype(jnp.bfloat16)

    x = x_ref[...]
    scores = (
        jnp.dot(x, rw_ref[...], preferred_element_type=jnp.float32)
        + rb_ref[...]
    )
    m = jnp.max(scores, axis=-1, keepdims=True)
    e = jnp.exp(scores - m)
    probs = e / jnp.sum(e, axis=-1, keepdims=True)

    i0 = jnp.argmax(probs, axis=-1).reshape(-1, 1)
    p0 = jnp.max(probs, axis=-1, keepdims=True)
    iota = jax.lax.broadcasted_iota(jnp.int32, probs.shape, 1)
    masked = jnp.where(iota == i0, probs - 2.0, probs)
    i1 = jnp.argmax(masked, axis=-1).reshape(-1, 1)
    p1 = jnp.max(masked, axis=-1, keepdims=True)

    denom = p0 + p1
    wmat = jnp.where(iota == i0, p0 / denom, 0.0) + jnp.where(
        iota == i1, p1 / denom, 0.0
    )

    xb = x.astype(jnp.bfloat16)
    acc = None
    for ei in range(_NUM_EXPERTS):
        w = wmat[:, ei].reshape(-1, 1)
        y = jnp.dot(xb, wb_ref[ei], preferred_element_type=jnp.float32)
        acc = w * y if acc is None else acc + w * y
    acc = acc + jnp.dot(wmat, eb_ref[...], preferred_element_type=jnp.float32)
    out_ref[...] = acc


def kernel(x, router_w, router_b, expert_w, expert_b):
    b, s, h = x.shape
    n_tok = b * s
    flat_x = x.reshape(n_tok, h)
    rb2 = router_b.reshape(1, -1)

    out = pl.pallas_call(
        _moe_tile_kernel,
        grid=(n_tok // _TILE,),
        in_specs=[
            pl.BlockSpec((_TILE, h), lambda t: (t, 0)),
            pl.BlockSpec((h, _NUM_EXPERTS), lambda t: (0, 0)),
            pl.BlockSpec((1, _NUM_EXPERTS), lambda t: (0, 0)),
            pl.BlockSpec((_NUM_EXPERTS, h, h), lambda t: (0, 0, 0)),
            pl.BlockSpec((_NUM_EXPERTS, h), lambda t: (0, 0)),
        ],
        out_specs=pl.BlockSpec((_TILE, h), lambda t: (t, 0)),
        out_shape=jax.ShapeDtypeStruct((n_tok, h), jnp.float32),
        scratch_shapes=[
            pltpu.VMEM((_NUM_EXPERTS, h, h), jnp.bfloat16),
        ],
    )(flat_x, router_w, rb2, expert_w, expert_b)
    return out.reshape(b, s, h)


# speedup vs baseline: 2.7206x; 1.0042x over previous
import jax
import jax.numpy as jnp
from jax.experimental import pallas as pl
from jax.experimental.pallas import tpu as pltpu

_NUM_EXPERTS = 8
_TILE = 1024


def _moe_tile_kernel(x_ref, rw_ref, rb_ref, ew_ref, eb_ref, out_ref, wb_ref):
    @pl.when(pl.program_id(0) == 0)
    def _cast_weights():
        wb_ref[...] = ew_ref[...].astype(jnp.bfloat16)

    x = x_ref[...]
    scores = (
        jnp.dot(x, rw_ref[...], preferred_element_type=jnp.float32)
        + rb_ref[...]
    )
    m = jnp.max(scores, axis=-1, keepdims=True)
    e = jnp.exp(scores - m)
    probs = e / jnp.sum(e, axis=-1, keepdims=True)

    i0 = jnp.argmax(probs, axis=-1).reshape(-1, 1)
    p0 = jnp.max(probs, axis=-1, keepdims=True)
    iota = jax.lax.broadcasted_iota(jnp.int32, probs.shape, 1)
    masked = jnp.where(iota == i0, probs - 2.0, probs)
    i1 = jnp.argmax(masked, axis=-1).reshape(-1, 1)
    p1 = jnp.max(masked, axis=-1, keepdims=True)

    denom = p0 + p1
    wmat = jnp.where(iota == i0, p0 / denom, 0.0) + jnp.where(
        iota == i1, p1 / denom, 0.0
    )

    xb = x.astype(jnp.bfloat16)
    acc = None
    for ei in range(_NUM_EXPERTS):
        w = wmat[:, ei].reshape(-1, 1)
        y = jnp.dot(xb, wb_ref[ei], preferred_element_type=jnp.float32)
        acc = w * y if acc is None else acc + w * y
    acc = acc + jnp.dot(wmat, eb_ref[...], preferred_element_type=jnp.float32)
    out_ref[...] = acc


def kernel(x, router_w, router_b, expert_w, expert_b):
    b, s, h = x.shape
    n_tok = b * s
    flat_x = x.reshape(n_tok, h)
    rb2 = router_b.reshape(1, -1)

    out = pl.pallas_call(
        _moe_tile_kernel,
        grid=(n_tok // _TILE,),
        in_specs=[
            pl.BlockSpec((_TILE, h), lambda t: (t, 0)),
            pl.BlockSpec((h, _NUM_EXPERTS), lambda t: (0, 0)),
            pl.BlockSpec((1, _NUM_EXPERTS), lambda t: (0, 0)),
            pl.BlockSpec((_NUM_EXPERTS, h, h), lambda t: (0, 0, 0)),
            pl.BlockSpec((_NUM_EXPERTS, h), lambda t: (0, 0)),
        ],
        out_specs=pl.BlockSpec((_TILE, h), lambda t: (t, 0)),
        out_shape=jax.ShapeDtypeStruct((n_tok, h), jnp.float32),
        scratch_shapes=[
            pltpu.VMEM((_NUM_EXPERTS, h, h), jnp.bfloat16),
        ],
    )(flat_x, router_w, rb2, expert_w, expert_b)
    return out.reshape(b, s, h)
